# baseline (speedup 1.0000x reference)
import functools

import jax
import jax.numpy as jnp
from jax import lax
from jax.experimental import pallas as pl
from jax.experimental.pallas import tpu as pltpu
from jax.experimental.pallas import tpu_sc as plsc

N = 10000
E = 320000
D = 128
H = 128
C = 40
CP = 48

NC = 2
NS = 16
CH = 80
EPAD = 327680
NW = NC * NS
EPW = EPAD // NW
TCHUNK = EPW // CH
NP = 10240
RPT = NP // NS


@functools.cache
def _sc_kernels():
    mesh = plsc.VectorSubcoreMesh(core_axis_name="c", subcore_axis_name="s",
                                  num_cores=NC, num_subcores=NS)
    params = pltpu.CompilerParams(use_tc_tiling_on_sc=False)

    @functools.partial(
        pl.kernel,
        out_type=jax.ShapeDtypeStruct((NC, NP, 8), jnp.float32),
        mesh=mesh,
        compiler_params=params,
        scratch_types=[
            pltpu.VMEM((TCHUNK * NC, CH), jnp.int32),
            pltpu.VMEM((CH, 8), jnp.float32),
            pltpu.VMEM_SHARED((NP, 8), jnp.float32),
        ],
    )
    def deg_kernel(dst_hbm, ones_hbm, zeros_hbm, out_hbm, d_v, ones_v, acc_sh):
        c = lax.axis_index("c")
        s = lax.axis_index("s")
        w = c * NS + s
        pltpu.sync_copy(zeros_hbm.at[pl.ds(s * RPT, RPT)], acc_sh.at[pl.ds(s * RPT, RPT)])
        pltpu.sync_copy(ones_hbm, ones_v)
        pltpu.sync_copy(dst_hbm.at[w], d_v)

        def body(j, carry):
            pltpu.sync_copy(ones_v, acc_sh.at[d_v.at[j]], add=True)
            return carry

        lax.fori_loop(0, TCHUNK * NC, body, 0)
        plsc.subcore_barrier()
        pltpu.sync_copy(acc_sh.at[pl.ds(s * RPT, RPT)], out_hbm.at[c, pl.ds(s * RPT, RPT)])

    def make_agg(F, nphase):
        PC = TCHUNK // nphase

        @functools.partial(
            pl.kernel,
            out_type=jax.ShapeDtypeStruct((NC, NP, F), jnp.float32),
            mesh=mesh,
            compiler_params=params,
            scratch_types=[
                pltpu.VMEM((PC, CH), jnp.int32),
                pltpu.VMEM((PC, CH), jnp.int32),
                pltpu.VMEM((CH, F), jnp.float32),
                pltpu.VMEM((CH, F), jnp.float32),
                pltpu.VMEM_SHARED((NP, F), jnp.float32),
                pltpu.SemaphoreType.DMA,
                pltpu.SemaphoreType.DMA,
            ],
        )
        def agg(g_hbm, src_hbm, dst_hbm, zeros_hbm, out_hbm,
                src_v, dst_v, buf_a, buf_b, acc_sh, sem_a, sem_b):
            c = lax.axis_index("c")
            s = lax.axis_index("s")
            w = s * NC + c
            pltpu.sync_copy(zeros_hbm.at[pl.ds(s * RPT, RPT)], acc_sh.at[pl.ds(s * RPT, RPT)])

            def phase(p, carry):
                pltpu.sync_copy(src_hbm.at[w, pl.ds(p * PC, PC)], src_v)
                pltpu.sync_copy(dst_hbm.at[w, pl.ds(p * PC, PC)], dst_v)
                pltpu.async_copy(g_hbm.at[src_v.at[0]], buf_a, sem_a)

                def body(i, carry2):
                    j0 = 2 * i
                    j1 = j0 + 1
                    pltpu.async_copy(g_hbm.at[src_v.at[j1]], buf_b, sem_b)
                    pltpu.make_async_copy(g_hbm.at[src_v.at[j0]], buf_a, sem_a).wait()
                    pltpu.sync_copy(buf_a, acc_sh.at[dst_v.at[j0]], add=True)
                    jn = jnp.minimum(j0 + 2, PC - 1)
                    pltpu.async_copy(g_hbm.at[src_v.at[jn]], buf_a, sem_a)
                    pltpu.make_async_copy(g_hbm.at[src_v.at[j1]], buf_b, sem_b).wait()
                    pltpu.sync_copy(buf_b, acc_sh.at[dst_v.at[j1]], add=True)
                    return carry2

                lax.fori_loop(0, PC // 2, body, 0)
                pltpu.make_async_copy(g_hbm.at[src_v.at[0]], buf_a, sem_a).wait()
                return carry

            lax.fori_loop(0, nphase, phase, 0)
            plsc.subcore_barrier()
            pltpu.sync_copy(acc_sh.at[pl.ds(s * RPT, RPT)], out_hbm.at[c, pl.ds(s * RPT, RPT)])

        return agg

    return deg_kernel, make_agg(H, 4), make_agg(CP, 1)


_BR = 1000
_BRP = 1024


def _stage_b_body(x1_ref, x2_ref, w1_ref, degcat_ref, g_ref):
    dc = degcat_ref[...]
    di1 = lax.rsqrt(dc[:, 0:1] + 1.0)
    di2 = lax.rsqrt(dc[:, 8:9] + 1.0)
    w1 = w1_ref[...]
    g_ref[0] = jnp.dot(x1_ref[...], w1, preferred_element_type=jnp.float32) * di1
    g_ref[1] = jnp.dot(x2_ref[...], w1, preferred_element_type=jnp.float32) * di2


def _stage_b(x1, x2, W1, degcat):
    grid = NP // _BRP
    row = lambda i: (i, 0)
    return pl.pallas_call(
        _stage_b_body,
        grid=(grid,),
        in_specs=[
            pl.BlockSpec((_BRP, D), row),
            pl.BlockSpec((_BRP, D), row),
            pl.BlockSpec((D, H), lambda i: (0, 0)),
            pl.BlockSpec((_BRP, 16), row),
        ],
        out_specs=pl.BlockSpec((2, _BRP, H), lambda i: (0, i, 0)),
        out_shape=jax.ShapeDtypeStruct((2, NP, H), jnp.float32),
    )(x1, x2, W1, degcat)


def _stage_d_body(g1_ref, a10_ref, a11_ref, g2_ref, a20_ref, a21_ref,
                  w2_ref, b1_ref, degcat_ref, gy_ref):
    dc = degcat_ref[...]
    di1 = lax.rsqrt(dc[:, 0:1] + 1.0)
    di2 = lax.rsqrt(dc[:, 8:9] + 1.0)
    w2 = w2_ref[...]
    b1 = b1_ref[...]
    h1 = jnp.maximum(di1 * (a10_ref[0] + a11_ref[0] + g1_ref[0]) + b1, 0.0)
    h2 = jnp.maximum(di2 * (a20_ref[0] + a21_ref[0] + g2_ref[0]) + b1, 0.0)
    gy_ref[0] = jnp.dot(h1, w2, preferred_element_type=jnp.float32) * di1
    gy_ref[1] = jnp.dot(h2, w2, preferred_element_type=jnp.float32) * di2


def _stage_d(g, a1, a2, W2p, b1r, degcat):
    grid = NP // _BRP
    g0 = lambda i: (0, i, 0)
    g1 = lambda i: (1, i, 0)
    fixed = lambda i: (0, 0)
    return pl.pallas_call(
        _stage_d_body,
        grid=(grid,),
        in_specs=[
            pl.BlockSpec((1, _BRP, H), g0),
            pl.BlockSpec((1, _BRP, H), g0),
            pl.BlockSpec((1, _BRP, H), g1),
            pl.BlockSpec((1, _BRP, H), g1),
            pl.BlockSpec((1, _BRP, H), g0),
            pl.BlockSpec((1, _BRP, H), g1),
            pl.BlockSpec((H, CP), fixed),
            pl.BlockSpec((1, H), fixed),
            pl.BlockSpec((_BRP, 16), lambda i: (i, 0)),
        ],
        out_specs=pl.BlockSpec((2, _BRP, CP), lambda i: (0, i, 0)),
        out_shape=jax.ShapeDtypeStruct((2, NP, CP), jnp.float32),
    )(g, a1, a1, g, a2, a2, W2p, b1r, degcat)


def _stage_f_body(gy1_ref, ay10_ref, ay11_ref, gy2_ref, ay20_ref, ay21_ref,
                  b2_ref, degcat_ref, ly_ref, lz_ref, cd_ref):
    dc = degcat_ref[...]
    di1 = lax.rsqrt(dc[:, 0:1] + 1.0)
    di2 = lax.rsqrt(dc[:, 8:9] + 1.0)
    b2 = b2_ref[...]
    y = di1 * (ay10_ref[0] + ay11_ref[0] + gy1_ref[0]) + b2
    z = di2 * (ay20_ref[0] + ay21_ref[0] + gy2_ref[0]) + b2
    ny = jnp.maximum(jnp.sqrt(jnp.sum(y * y, axis=1, keepdims=True)), 1e-8)
    nz = jnp.maximum(jnp.sqrt(jnp.sum(z * z, axis=1, keepdims=True)), 1e-8)
    cos = jnp.sum(y * z, axis=1, keepdims=True) / (ny * nz)
    cd_ref[...] = 1.0 - cos
    mask = lax.broadcasted_iota(jnp.int32, (_BR, CP), 1) < C
    neg = jnp.float32(-1e30)
    my = jnp.max(jnp.where(mask, y, neg), axis=1, keepdims=True)
    mz = jnp.max(jnp.where(mask, z, neg), axis=1, keepdims=True)
    lse_y = jnp.log(jnp.sum(jnp.where(mask, jnp.exp(y - my), 0.0), axis=1, keepdims=True))
    lse_z = jnp.log(jnp.sum(jnp.where(mask, jnp.exp(z - mz), 0.0), axis=1, keepdims=True))
    ly_ref[...] = y - my - lse_y
    lz_ref[...] = z - mz - lse_z


def _stage_f(gy, ay1, ay2, b2p, degcat):
    grid = N // _BR
    g0 = lambda i: (0, i, 0)
    g1 = lambda i: (1, i, 0)
    row = lambda i: (i, 0)
    fixed = lambda i: (0, 0)
    return pl.pallas_call(
        _stage_f_body,
        grid=(grid,),
        in_specs=[
            pl.BlockSpec((1, _BR, CP), g0),
            pl.BlockSpec((1, _BR, CP), g0),
            pl.BlockSpec((1, _BR, CP), g1),
            pl.BlockSpec((1, _BR, CP), g1),
            pl.BlockSpec((1, _BR, CP), g0),
            pl.BlockSpec((1, _BR, CP), g1),
            pl.BlockSpec((1, CP), fixed),
            pl.BlockSpec((_BR, 16), row),
        ],
        out_specs=[
            pl.BlockSpec((_BR, CP), row),
            pl.BlockSpec((_BR, CP), row),
            pl.BlockSpec((_BR, 1), row),
        ],
        out_shape=[
            jax.ShapeDtypeStruct((N, CP), jnp.float32),
            jax.ShapeDtypeStruct((N, CP), jnp.float32),
            jax.ShapeDtypeStruct((N, 1), jnp.float32),
        ],
    )(gy, ay1, ay1, gy, ay2, ay2, b2p, degcat)


def kernel(x1, edge_index1, x2, edge_index2, W1, b1, W2, b2):
    padn = EPAD - E
    pad_src = jnp.full((padn,), N, jnp.int32)
    pad_dst = N + (jnp.arange(padn, dtype=jnp.int32) % (NP - N))
    pad = jnp.stack([pad_src, pad_dst])
    e1 = jnp.concatenate([edge_index1, pad], axis=1)
    e2 = jnp.concatenate([edge_index2, pad], axis=1)
    src1 = e1[0].reshape(NW, TCHUNK, CH)
    dst1 = e1[1].reshape(NW, TCHUNK, CH)
    src2 = e2[0].reshape(NW, TCHUNK, CH) + NP
    dst2 = e2[1].reshape(NW, TCHUNK, CH)
    dst_deg = jnp.stack([e1[1].reshape(NS, TCHUNK * NC, CH),
                         e2[1].reshape(NS, TCHUNK * NC, CH)]).reshape(NC * NS, TCHUNK * NC, CH)

    ones8 = jnp.ones((CH, 8), jnp.float32)
    zeros8 = jnp.zeros((NP, 8), jnp.float32)
    zeros_h = jnp.zeros((NP, H), jnp.float32)
    zeros_c = jnp.zeros((NP, CP), jnp.float32)

    W2p = jnp.zeros((H, CP), jnp.float32).at[:, :C].set(W2)
    b1r = b1.reshape(1, H)
    b2p = jnp.zeros((1, CP), jnp.float32).at[0, :C].set(b2)

    _deg_kernel, _agg_h, _agg_c = _sc_kernels()

    degp = _deg_kernel(dst_deg, ones8, zeros8)
    degcat = jnp.concatenate([degp[0, :N], degp[1, :N]], axis=1)

    g = _stage_b(x1, x2, W1, degcat)
    gflat = g.reshape(2 * NP, H)
    a1 = _agg_h(gflat, src1, dst1, zeros_h)
    a2 = _agg_h(gflat, src2, dst2, zeros_h)
    gy = _stage_d(g, a1, a2, W2p, b1r, degcat)
    gyflat = gy.reshape(2 * NP, CP)
    ay1 = _agg_c(gyflat, src1, dst1, zeros_c)
    ay2 = _agg_c(gyflat, src2, dst2, zeros_c)
    ly64, lz64, cd = _stage_f(gy, ay1, ay2, b2p, degcat)

    ly = ly64[:, :C]
    lz = lz64[:, :C]
    return (ly, cd[:, 0], lz, ly, ly)


# speedup vs baseline: 1.9046x; 1.9046x over previous
import functools

import jax
import jax.numpy as jnp
from jax import lax
from jax.experimental import pallas as pl
from jax.experimental.pallas import tpu as pltpu
from jax.experimental.pallas import tpu_sc as plsc

N = 10000
E = 320000
D = 128
H = 128
C = 40
CP = 48

NC = 2
NS = 16
CH = 80
NW = NC * NS
EPW = E // NW
TCHUNK = EPW // CH
NP = 10240
RPT = NP // NS


@functools.cache
def _sc_kernels():
    mesh = plsc.VectorSubcoreMesh(core_axis_name="c", subcore_axis_name="s",
                                  num_cores=NC, num_subcores=NS)
    params = pltpu.CompilerParams(use_tc_tiling_on_sc=False)

    @functools.partial(
        pl.kernel,
        out_type=jax.ShapeDtypeStruct((NC, NP, 8), jnp.float32),
        mesh=mesh,
        compiler_params=params,
        scratch_types=[
            pltpu.VMEM((TCHUNK * NC, CH), jnp.int32),
            pltpu.VMEM((CH, 8), jnp.float32),
            pltpu.VMEM_SHARED((NP, 8), jnp.float32),
        ],
    )
    def deg_kernel(dst_hbm, ones_hbm, zeros_hbm, out_hbm, d_v, ones_v, acc_sh):
        c = lax.axis_index("c")
        s = lax.axis_index("s")
        w = c * NS + s
        pltpu.sync_copy(zeros_hbm.at[pl.ds(s * RPT, RPT)], acc_sh.at[pl.ds(s * RPT, RPT)])
        pltpu.sync_copy(ones_hbm, ones_v)
        pltpu.sync_copy(dst_hbm.at[w], d_v)

        def body(j, carry):
            pltpu.sync_copy(ones_v, acc_sh.at[d_v.at[j]], add=True)
            return carry

        lax.fori_loop(0, TCHUNK * NC, body, 0)
        plsc.subcore_barrier()
        pltpu.sync_copy(acc_sh.at[pl.ds(s * RPT, RPT)], out_hbm.at[c, pl.ds(s * RPT, RPT)])

    def make_agg(F):
        @functools.partial(
            pl.kernel,
            out_type=jax.ShapeDtypeStruct((NC, NP, F), jnp.float32),
            mesh=mesh,
            compiler_params=params,
            scratch_types=[
                pltpu.VMEM((TCHUNK, CH), jnp.int32),
                pltpu.VMEM((TCHUNK, CH), jnp.int32),
                pltpu.VMEM((CH, F), jnp.float32),
                pltpu.VMEM_SHARED((NP, F), jnp.float32),
                pltpu.SemaphoreType.DMA,
            ],
        )
        def agg(g_hbm, src_hbm, dst_hbm, zeros_hbm, out_hbm,
                src_v, dst_v, buf_a, acc_sh, sem_a):
            c = lax.axis_index("c")
            s = lax.axis_index("s")
            w = s * NC + c
            pltpu.sync_copy(zeros_hbm.at[pl.ds(s * RPT, RPT)], acc_sh.at[pl.ds(s * RPT, RPT)])

            pltpu.sync_copy(src_hbm.at[w], src_v)
            pltpu.sync_copy(dst_hbm.at[w], dst_v)

            def body(j, carry):
                pltpu.async_copy(g_hbm.at[src_v.at[j]], buf_a, sem_a).wait()
                pltpu.sync_copy(buf_a, acc_sh.at[dst_v.at[j]], add=True)
                return carry

            lax.fori_loop(0, TCHUNK, body, 0)
            plsc.subcore_barrier()
            pltpu.sync_copy(acc_sh.at[pl.ds(s * RPT, RPT)], out_hbm.at[c, pl.ds(s * RPT, RPT)])

        return agg

    return deg_kernel, make_agg(H), make_agg(CP)


_BR = 1000
_BRP = 1024


def _stage_b_body(x1_ref, x2_ref, w1_ref, degcat_ref, g_ref):
    dc = degcat_ref[...]
    di1 = lax.rsqrt(dc[:, 0:1] + 1.0)
    di2 = lax.rsqrt(dc[:, 8:9] + 1.0)
    w1 = w1_ref[...]
    g_ref[0] = jnp.dot(x1_ref[...], w1, preferred_element_type=jnp.float32) * di1
    g_ref[1] = jnp.dot(x2_ref[...], w1, preferred_element_type=jnp.float32) * di2


def _stage_b(x1, x2, W1, degcat):
    grid = NP // _BRP
    row = lambda i: (i, 0)
    return pl.pallas_call(
        _stage_b_body,
        grid=(grid,),
        in_specs=[
            pl.BlockSpec((_BRP, D), row),
            pl.BlockSpec((_BRP, D), row),
            pl.BlockSpec((D, H), lambda i: (0, 0)),
            pl.BlockSpec((_BRP, 16), row),
        ],
        out_specs=pl.BlockSpec((2, _BRP, H), lambda i: (0, i, 0)),
        out_shape=jax.ShapeDtypeStruct((2, NP, H), jnp.float32),
    )(x1, x2, W1, degcat)


def _stage_d_body(g1_ref, a10_ref, a11_ref, g2_ref, a20_ref, a21_ref,
                  w2_ref, b1_ref, degcat_ref, gy_ref):
    dc = degcat_ref[...]
    di1 = lax.rsqrt(dc[:, 0:1] + 1.0)
    di2 = lax.rsqrt(dc[:, 8:9] + 1.0)
    w2 = w2_ref[...]
    b1 = b1_ref[...]
    h1 = jnp.maximum(di1 * (a10_ref[0] + a11_ref[0] + g1_ref[0]) + b1, 0.0)
    h2 = jnp.maximum(di2 * (a20_ref[0] + a21_ref[0] + g2_ref[0]) + b1, 0.0)
    gy_ref[0] = jnp.dot(h1, w2, preferred_element_type=jnp.float32) * di1
    gy_ref[1] = jnp.dot(h2, w2, preferred_element_type=jnp.float32) * di2


def _stage_d(g, a1, a2, W2p, b1r, degcat):
    grid = NP // _BRP
    g0 = lambda i: (0, i, 0)
    g1 = lambda i: (1, i, 0)
    fixed = lambda i: (0, 0)
    return pl.pallas_call(
        _stage_d_body,
        grid=(grid,),
        in_specs=[
            pl.BlockSpec((1, _BRP, H), g0),
            pl.BlockSpec((1, _BRP, H), g0),
            pl.BlockSpec((1, _BRP, H), g1),
            pl.BlockSpec((1, _BRP, H), g1),
            pl.BlockSpec((1, _BRP, H), g0),
            pl.BlockSpec((1, _BRP, H), g1),
            pl.BlockSpec((H, CP), fixed),
            pl.BlockSpec((1, H), fixed),
            pl.BlockSpec((_BRP, 16), lambda i: (i, 0)),
        ],
        out_specs=pl.BlockSpec((2, _BRP, CP), lambda i: (0, i, 0)),
        out_shape=jax.ShapeDtypeStruct((2, NP, CP), jnp.float32),
    )(g, a1, a1, g, a2, a2, W2p, b1r, degcat)


def _stage_f_body(gy1_ref, ay10_ref, ay11_ref, gy2_ref, ay20_ref, ay21_ref,
                  b2_ref, degcat_ref, ly_ref, lz_ref, cd_ref):
    dc = degcat_ref[...]
    di1 = lax.rsqrt(dc[:, 0:1] + 1.0)
    di2 = lax.rsqrt(dc[:, 8:9] + 1.0)
    b2 = b2_ref[...]
    y = di1 * (ay10_ref[0] + ay11_ref[0] + gy1_ref[0]) + b2
    z = di2 * (ay20_ref[0] + ay21_ref[0] + gy2_ref[0]) + b2
    ny = jnp.maximum(jnp.sqrt(jnp.sum(y * y, axis=1, keepdims=True)), 1e-8)
    nz = jnp.maximum(jnp.sqrt(jnp.sum(z * z, axis=1, keepdims=True)), 1e-8)
    cos = jnp.sum(y * z, axis=1, keepdims=True) / (ny * nz)
    cd_ref[...] = 1.0 - cos
    mask = lax.broadcasted_iota(jnp.int32, (_BR, CP), 1) < C
    neg = jnp.float32(-1e30)
    my = jnp.max(jnp.where(mask, y, neg), axis=1, keepdims=True)
    mz = jnp.max(jnp.where(mask, z, neg), axis=1, keepdims=True)
    lse_y = jnp.log(jnp.sum(jnp.where(mask, jnp.exp(y - my), 0.0), axis=1, keepdims=True))
    lse_z = jnp.log(jnp.sum(jnp.where(mask, jnp.exp(z - mz), 0.0), axis=1, keepdims=True))
    ly_ref[...] = y - my - lse_y
    lz_ref[...] = z - mz - lse_z


def _stage_f(gy, ay1, ay2, b2p, degcat):
    grid = N // _BR
    g0 = lambda i: (0, i, 0)
    g1 = lambda i: (1, i, 0)
    row = lambda i: (i, 0)
    fixed = lambda i: (0, 0)
    return pl.pallas_call(
        _stage_f_body,
        grid=(grid,),
        in_specs=[
            pl.BlockSpec((1, _BR, CP), g0),
            pl.BlockSpec((1, _BR, CP), g0),
            pl.BlockSpec((1, _BR, CP), g1),
            pl.BlockSpec((1, _BR, CP), g1),
            pl.BlockSpec((1, _BR, CP), g0),
            pl.BlockSpec((1, _BR, CP), g1),
            pl.BlockSpec((1, CP), fixed),
            pl.BlockSpec((_BR, 16), row),
        ],
        out_specs=[
            pl.BlockSpec((_BR, CP), row),
            pl.BlockSpec((_BR, CP), row),
            pl.BlockSpec((_BR, 1), row),
        ],
        out_shape=[
            jax.ShapeDtypeStruct((N, CP), jnp.float32),
            jax.ShapeDtypeStruct((N, CP), jnp.float32),
            jax.ShapeDtypeStruct((N, 1), jnp.float32),
        ],
    )(gy, ay1, ay1, gy, ay2, ay2, b2p, degcat)


def kernel(x1, edge_index1, x2, edge_index2, W1, b1, W2, b2):
    src1 = edge_index1[0].reshape(NW, TCHUNK, CH)
    dst1 = edge_index1[1].reshape(NW, TCHUNK, CH)
    src2 = edge_index2[0].reshape(NW, TCHUNK, CH) + NP
    dst2 = edge_index2[1].reshape(NW, TCHUNK, CH)
    dst_deg = jnp.stack([edge_index1[1].reshape(NS, TCHUNK * NC, CH),
                         edge_index2[1].reshape(NS, TCHUNK * NC, CH)]).reshape(NC * NS, TCHUNK * NC, CH)

    ones8 = jnp.ones((CH, 8), jnp.float32)
    zeros8 = jnp.zeros((NP, 8), jnp.float32)
    zeros_h = jnp.zeros((NP, H), jnp.float32)
    zeros_c = jnp.zeros((NP, CP), jnp.float32)

    W2p = jnp.zeros((H, CP), jnp.float32).at[:, :C].set(W2)
    b1r = b1.reshape(1, H)
    b2p = jnp.zeros((1, CP), jnp.float32).at[0, :C].set(b2)

    _deg_kernel, _agg_h, _agg_c = _sc_kernels()

    degp = _deg_kernel(dst_deg, ones8, zeros8)
    degcat = jnp.concatenate([degp[0, :N], degp[1, :N]], axis=1)

    g = _stage_b(x1, x2, W1, degcat)
    gflat = g.reshape(2 * NP, H)
    a1 = _agg_h(gflat, src1, dst1, zeros_h)
    a2 = _agg_h(gflat, src2, dst2, zeros_h)
    gy = _stage_d(g, a1, a2, W2p, b1r, degcat)
    gyflat = gy.reshape(2 * NP, CP)
    ay1 = _agg_c(gyflat, src1, dst1, zeros_c)
    ay2 = _agg_c(gyflat, src2, dst2, zeros_c)
    ly64, lz64, cd = _stage_f(gy, ay1, ay2, b2p, degcat)

    ly = ly64[:, :C]
    lz = lz64[:, :C]
    return (ly, cd[:, 0], lz, ly, ly)


# speedup vs baseline: 2.8601x; 1.5017x over previous
import functools

import jax
import jax.numpy as jnp
from jax import lax
from jax.experimental import pallas as pl
from jax.experimental.pallas import tpu as pltpu
from jax.experimental.pallas import tpu_sc as plsc

N = 10000
E = 320000
D = 128
H = 128
C = 40
CP = 48

NC = 2
NS = 16
CH = 80
NW = NC * NS
EPW = E // NW
TCHUNK = EPW // CH
NP = 10240
RPT = NP // NS


@functools.cache
def _sc_kernels():
    mesh = plsc.VectorSubcoreMesh(core_axis_name="c", subcore_axis_name="s",
                                  num_cores=NC, num_subcores=NS)
    params = pltpu.CompilerParams(use_tc_tiling_on_sc=False)

    @functools.partial(
        pl.kernel,
        out_type=jax.ShapeDtypeStruct((NC, NP, 8), jnp.float32),
        mesh=mesh,
        compiler_params=params,
        scratch_types=[
            pltpu.VMEM((TCHUNK * NC, CH), jnp.int32),
            pltpu.VMEM((CH, 8), jnp.float32),
            pltpu.VMEM_SHARED((NP, 8), jnp.float32),
        ],
    )
    def deg_kernel(dst_hbm, ones_hbm, zeros_hbm, out_hbm, d_v, ones_v, acc_sh):
        c = lax.axis_index("c")
        s = lax.axis_index("s")
        w = c * NS + s
        pltpu.sync_copy(zeros_hbm.at[pl.ds(s * RPT, RPT)], acc_sh.at[pl.ds(s * RPT, RPT)])
        pltpu.sync_copy(ones_hbm, ones_v)
        pltpu.sync_copy(dst_hbm.at[w], d_v)

        def body(j, carry):
            pltpu.sync_copy(ones_v, acc_sh.at[d_v.at[j]], add=True)
            return carry

        lax.fori_loop(0, TCHUNK * NC, body, 0)
        plsc.subcore_barrier()
        pltpu.sync_copy(acc_sh.at[pl.ds(s * RPT, RPT)], out_hbm.at[c, pl.ds(s * RPT, RPT)])

    def make_agg(F):
        @functools.partial(
            pl.kernel,
            out_type=jax.ShapeDtypeStruct((NC, NP, F), jnp.float32),
            mesh=mesh,
            compiler_params=params,
            scratch_types=[
                pltpu.VMEM((TCHUNK, CH), jnp.int32),
                pltpu.VMEM((TCHUNK, CH), jnp.int32),
                pltpu.VMEM((CH, F), jnp.float32),
                pltpu.VMEM((CH, F), jnp.float32),
                pltpu.VMEM_SHARED((NP, F), jnp.float32),
                pltpu.SemaphoreType.DMA,
                pltpu.SemaphoreType.DMA,
            ],
        )
        def agg(g_hbm, src_hbm, dst_hbm, zeros_hbm, out_hbm,
                src_v, dst_v, buf_a, buf_b, acc_sh, sem_a, sem_b):
            c = lax.axis_index("c")
            s = lax.axis_index("s")
            w = s * NC + c
            pltpu.sync_copy(zeros_hbm.at[pl.ds(s * RPT, RPT)], acc_sh.at[pl.ds(s * RPT, RPT)])

            pltpu.sync_copy(src_hbm.at[w], src_v)
            pltpu.sync_copy(dst_hbm.at[w], dst_v)

            pltpu.async_copy(g_hbm.at[src_v.at[0]], buf_a, sem_a)

            def body(i, carry):
                j0 = 2 * i
                j1 = j0 + 1
                pltpu.async_copy(g_hbm.at[src_v.at[j1]], buf_b, sem_b)
                pltpu.make_async_copy(g_hbm.at[src_v.at[j0]], buf_a, sem_a).wait()
                pltpu.sync_copy(buf_a, acc_sh.at[dst_v.at[j0]], add=True)
                pltpu.async_copy(g_hbm.at[src_v.at[j0 + 2]], buf_a, sem_a)
                pltpu.make_async_copy(g_hbm.at[src_v.at[j1]], buf_b, sem_b).wait()
                pltpu.sync_copy(buf_b, acc_sh.at[dst_v.at[j1]], add=True)
                return carry

            lax.fori_loop(0, TCHUNK // 2, body, 0)
            pltpu.make_async_copy(g_hbm.at[src_v.at[0]], buf_a, sem_a).wait()
            pltpu.sync_copy(buf_a, acc_sh.at[dst_v.at[TCHUNK - 1]], add=True)
            plsc.subcore_barrier()
            pltpu.sync_copy(acc_sh.at[pl.ds(s * RPT, RPT)], out_hbm.at[c, pl.ds(s * RPT, RPT)])

        return agg

    return deg_kernel, make_agg(H), make_agg(CP)


_BR = 1000
_BRP = 1024


def _stage_b_body(x1_ref, x2_ref, w1_ref, degcat_ref, g_ref):
    dc = degcat_ref[...]
    di1 = lax.rsqrt(dc[:, 0:1] + 1.0)
    di2 = lax.rsqrt(dc[:, 8:9] + 1.0)
    w1 = w1_ref[...]
    g_ref[0] = jnp.dot(x1_ref[...], w1, preferred_element_type=jnp.float32) * di1
    g_ref[1] = jnp.dot(x2_ref[...], w1, preferred_element_type=jnp.float32) * di2


def _stage_b(x1, x2, W1, degcat):
    grid = NP // _BRP
    row = lambda i: (i, 0)
    return pl.pallas_call(
        _stage_b_body,
        grid=(grid,),
        in_specs=[
            pl.BlockSpec((_BRP, D), row),
            pl.BlockSpec((_BRP, D), row),
            pl.BlockSpec((D, H), lambda i: (0, 0)),
            pl.BlockSpec((_BRP, 16), row),
        ],
        out_specs=pl.BlockSpec((2, _BRP, H), lambda i: (0, i, 0)),
        out_shape=jax.ShapeDtypeStruct((2, NP, H), jnp.float32),
    )(x1, x2, W1, degcat)


def _stage_d_body(g1_ref, a10_ref, a11_ref, g2_ref, a20_ref, a21_ref,
                  w2_ref, b1_ref, degcat_ref, gy_ref):
    dc = degcat_ref[...]
    di1 = lax.rsqrt(dc[:, 0:1] + 1.0)
    di2 = lax.rsqrt(dc[:, 8:9] + 1.0)
    w2 = w2_ref[...]
    b1 = b1_ref[...]
    h1 = jnp.maximum(di1 * (a10_ref[0] + a11_ref[0] + g1_ref[0]) + b1, 0.0)
    h2 = jnp.maximum(di2 * (a20_ref[0] + a21_ref[0] + g2_ref[0]) + b1, 0.0)
    gy_ref[0] = jnp.dot(h1, w2, preferred_element_type=jnp.float32) * di1
    gy_ref[1] = jnp.dot(h2, w2, preferred_element_type=jnp.float32) * di2


def _stage_d(g, a1, a2, W2p, b1r, degcat):
    grid = NP // _BRP
    g0 = lambda i: (0, i, 0)
    g1 = lambda i: (1, i, 0)
    fixed = lambda i: (0, 0)
    return pl.pallas_call(
        _stage_d_body,
        grid=(grid,),
        in_specs=[
            pl.BlockSpec((1, _BRP, H), g0),
            pl.BlockSpec((1, _BRP, H), g0),
            pl.BlockSpec((1, _BRP, H), g1),
            pl.BlockSpec((1, _BRP, H), g1),
            pl.BlockSpec((1, _BRP, H), g0),
            pl.BlockSpec((1, _BRP, H), g1),
            pl.BlockSpec((H, CP), fixed),
            pl.BlockSpec((1, H), fixed),
            pl.BlockSpec((_BRP, 16), lambda i: (i, 0)),
        ],
        out_specs=pl.BlockSpec((2, _BRP, CP), lambda i: (0, i, 0)),
        out_shape=jax.ShapeDtypeStruct((2, NP, CP), jnp.float32),
    )(g, a1, a1, g, a2, a2, W2p, b1r, degcat)


def _stage_f_body(gy1_ref, ay10_ref, ay11_ref, gy2_ref, ay20_ref, ay21_ref,
                  b2_ref, degcat_ref, ly_ref, lz_ref, cd_ref):
    dc = degcat_ref[...]
    di1 = lax.rsqrt(dc[:, 0:1] + 1.0)
    di2 = lax.rsqrt(dc[:, 8:9] + 1.0)
    b2 = b2_ref[...]
    y = di1 * (ay10_ref[0] + ay11_ref[0] + gy1_ref[0]) + b2
    z = di2 * (ay20_ref[0] + ay21_ref[0] + gy2_ref[0]) + b2
    ny = jnp.maximum(jnp.sqrt(jnp.sum(y * y, axis=1, keepdims=True)), 1e-8)
    nz = jnp.maximum(jnp.sqrt(jnp.sum(z * z, axis=1, keepdims=True)), 1e-8)
    cos = jnp.sum(y * z, axis=1, keepdims=True) / (ny * nz)
    cd_ref[...] = 1.0 - cos
    mask = lax.broadcasted_iota(jnp.int32, (_BR, CP), 1) < C
    neg = jnp.float32(-1e30)
    my = jnp.max(jnp.where(mask, y, neg), axis=1, keepdims=True)
    mz = jnp.max(jnp.where(mask, z, neg), axis=1, keepdims=True)
    lse_y = jnp.log(jnp.sum(jnp.where(mask, jnp.exp(y - my), 0.0), axis=1, keepdims=True))
    lse_z = jnp.log(jnp.sum(jnp.where(mask, jnp.exp(z - mz), 0.0), axis=1, keepdims=True))
    ly_ref[...] = y - my - lse_y
    lz_ref[...] = z - mz - lse_z


def _stage_f(gy, ay1, ay2, b2p, degcat):
    grid = N // _BR
    g0 = lambda i: (0, i, 0)
    g1 = lambda i: (1, i, 0)
    row = lambda i: (i, 0)
    fixed = lambda i: (0, 0)
    return pl.pallas_call(
        _stage_f_body,
        grid=(grid,),
        in_specs=[
            pl.BlockSpec((1, _BR, CP), g0),
            pl.BlockSpec((1, _BR, CP), g0),
            pl.BlockSpec((1, _BR, CP), g1),
            pl.BlockSpec((1, _BR, CP), g1),
            pl.BlockSpec((1, _BR, CP), g0),
            pl.BlockSpec((1, _BR, CP), g1),
            pl.BlockSpec((1, CP), fixed),
            pl.BlockSpec((_BR, 16), row),
        ],
        out_specs=[
            pl.BlockSpec((_BR, CP), row),
            pl.BlockSpec((_BR, CP), row),
            pl.BlockSpec((_BR, 1), row),
        ],
        out_shape=[
            jax.ShapeDtypeStruct((N, CP), jnp.float32),
            jax.ShapeDtypeStruct((N, CP), jnp.float32),
            jax.ShapeDtypeStruct((N, 1), jnp.float32),
        ],
    )(gy, ay1, ay1, gy, ay2, ay2, b2p, degcat)


def kernel(x1, edge_index1, x2, edge_index2, W1, b1, W2, b2):
    src1 = edge_index1[0].reshape(NW, TCHUNK, CH)
    dst1 = edge_index1[1].reshape(NW, TCHUNK, CH)
    src2 = edge_index2[0].reshape(NW, TCHUNK, CH) + NP
    dst2 = edge_index2[1].reshape(NW, TCHUNK, CH)
    dst_deg = jnp.stack([edge_index1[1].reshape(NS, TCHUNK * NC, CH),
                         edge_index2[1].reshape(NS, TCHUNK * NC, CH)]).reshape(NC * NS, TCHUNK * NC, CH)

    ones8 = jnp.ones((CH, 8), jnp.float32)
    zeros8 = jnp.zeros((NP, 8), jnp.float32)
    zeros_h = jnp.zeros((NP, H), jnp.float32)
    zeros_c = jnp.zeros((NP, CP), jnp.float32)

    W2p = jnp.zeros((H, CP), jnp.float32).at[:, :C].set(W2)
    b1r = b1.reshape(1, H)
    b2p = jnp.zeros((1, CP), jnp.float32).at[0, :C].set(b2)

    _deg_kernel, _agg_h, _agg_c = _sc_kernels()

    degp = _deg_kernel(dst_deg, ones8, zeros8)
    degcat = jnp.concatenate([degp[0, :N], degp[1, :N]], axis=1)

    g = _stage_b(x1, x2, W1, degcat)
    gflat = g.reshape(2 * NP, H)
    a1 = _agg_h(gflat, src1, dst1, zeros_h)
    a2 = _agg_h(gflat, src2, dst2, zeros_h)
    gy = _stage_d(g, a1, a2, W2p, b1r, degcat)
    gyflat = gy.reshape(2 * NP, CP)
    ay1 = _agg_c(gyflat, src1, dst1, zeros_c)
    ay2 = _agg_c(gyflat, src2, dst2, zeros_c)
    ly64, lz64, cd = _stage_f(gy, ay1, ay2, b2p, degcat)

    ly = ly64[:, :C]
    lz = lz64[:, :C]
    return (ly, cd[:, 0], lz, ly, ly)
